# Initial kernel scaffold; baseline (speedup 1.0000x reference)
import jax
import jax.numpy as jnp
from jax.experimental import pallas as pl


def kernel(features, af_W1, af_b1, af_W2, af_b2, as_W1, as_b1, as_W2, as_b2, at1_W, at2_W, at2_b, v_W, v_b):
    raise NotImplementedError("write your pallas kernel here")


# speedup vs baseline: 1.1495x; 1.1495x over previous
import jax
import jax.numpy as jnp
from jax import lax
from jax.experimental import pallas as pl
from jax.experimental.pallas import tpu as pltpu

F = 768
N = 8192
VF = 64
TILE = 512
NT = N // TILE
ACT = 2 * N + 2


def _fused_body(x_ref, afW1_ref, afb1_ref, afW2r_ref, afb2_ref,
                asW1b_ref, asb1_ref, at1W_ref, at2W_ref, at2b_ref,
                vW_ref, vb_ref, g1_ref, g3_ref,
                base2_ref, critic_ref, af_ref, at_ref,
                lg_scr, pooled_scr):
    i = pl.program_id(0)
    x = x_ref[...]

    h1 = jnp.maximum(x @ afW1_ref[...] + afb1_ref[...], 0.0)
    lg_tile = lax.dot_general(afW2r_ref[...], h1, (((1,), (1,)), ((), ())))
    lg_scr[:, pl.ds(i * TILE, TILE)] = lg_tile

    base2_ref[...] = x @ asW1b_ref[...] + asb1_ref[...]

    ht = jnp.maximum(x @ at1W_ref[...], 0.0)
    psum = jnp.sum(ht, axis=0, keepdims=True)

    @pl.when(i == 0)
    def _():
        pooled_scr[...] = jnp.zeros_like(pooled_scr)

    pooled_scr[...] += psum

    critic_ref[...] = jnp.maximum(x @ vW_ref[...] + vb_ref[...], 0.0)

    @pl.when(i == NT - 1)
    def _():
        z = lg_scr[...] + afb2_ref[0, 0] + g1_ref[...]
        m = jnp.max(z)
        idx = lax.broadcasted_iota(jnp.int32, (1, N), 1)
        af_ref[0, 0] = jnp.min(jnp.where(z == m, idx, N))

        lgt = pooled_scr[...] @ at2W_ref[...] + at2b_ref[...]
        zt = lgt + g3_ref[...]
        at_ref[0, 0] = jnp.where(zt[0, 1] > zt[0, 0], 1, 0)


def _as_body(base2_ref, row0_ref, asW2r_ref, asb2_ref, g2_ref, af_ref,
             as_ref, lg_scr):
    i = pl.program_id(0)
    c = af_ref[0, 0].astype(jnp.float32)
    h2 = jnp.maximum(base2_ref[...] + c * row0_ref[...], 0.0)
    lg_tile = lax.dot_general(asW2r_ref[...], h2, (((1,), (1,)), ((), ())))
    lg_scr[:, pl.ds(i * TILE, TILE)] = lg_tile

    @pl.when(i == NT - 1)
    def _():
        idx = lax.broadcasted_iota(jnp.int32, (1, N), 1)
        z = jnp.where(idx == af_ref[0, 0], -jnp.inf,
                      lg_scr[...] + asb2_ref[0, 0] + g2_ref[...])
        m = jnp.max(z)
        as_ref[0, 0] = jnp.min(jnp.where(z == m, idx, N))


def _actor_body(af_ref, as_ref, at_ref, out_ref):
    f1 = jnp.where(af_ref[0, 0] == 0, 1.0, 0.0).astype(jnp.float32)
    f2 = jnp.where(as_ref[0, 0] == 0, 1.0, 0.0).astype(jnp.float32)
    f3 = jnp.where(at_ref[0, 0] == 0, 1.0, 0.0).astype(jnp.float32)
    idx = lax.broadcasted_iota(jnp.int32, (1, ACT), 1)
    out_ref[...] = jnp.where(idx < N, f1, jnp.where(idx < 2 * N, f2, f3))


def _full(shape):
    return pl.BlockSpec(shape, lambda i: tuple(0 for _ in shape))


def _smem11():
    return pl.BlockSpec((1, 1), lambda i: (0, 0), memory_space=pltpu.SMEM)


def kernel(features, af_W1, af_b1, af_W2, af_b2, as_W1, as_b1, as_W2,
           as_b2, at1_W, at2_W, at2_b, v_W, v_b):
    key = jax.random.key(42)
    k1, k2, k3 = jax.random.split(key, 3)
    g1 = jax.random.gumbel(k1, (1, N), jnp.float32)
    g2 = jax.random.gumbel(k2, (1, N), jnp.float32)
    g3 = jax.random.gumbel(k3, (1, 2), jnp.float32)

    x = features.reshape(N, F)
    afb1 = af_b1.reshape(1, F)
    afW2r = af_W2.reshape(1, F)
    afb2 = af_b2.reshape(1, 1)
    asW1b = as_W1[1:]
    row0 = as_W1[0].reshape(1, F)
    asb1 = as_b1.reshape(1, F)
    asW2r = as_W2.reshape(1, F)
    asb2 = as_b2.reshape(1, 1)
    at2b = at2_b.reshape(1, 2)
    vb = v_b.reshape(1, VF)

    base2, critic, a_f, a_t = pl.pallas_call(
        _fused_body,
        grid=(NT,),
        in_specs=[
            pl.BlockSpec((TILE, F), lambda i: (i, 0)),
            _full((F, F)), _full((1, F)), _full((1, F)), _full((1, 1)),
            _full((F, F)), _full((1, F)),
            _full((F, F)), _full((F, 2)), _full((1, 2)),
            _full((F, VF)), _full((1, VF)),
            _full((1, N)), _full((1, 2)),
        ],
        out_specs=[
            pl.BlockSpec((TILE, F), lambda i: (i, 0)),
            pl.BlockSpec((TILE, VF), lambda i: (i, 0)),
            _smem11(), _smem11(),
        ],
        out_shape=[
            jax.ShapeDtypeStruct((N, F), jnp.float32),
            jax.ShapeDtypeStruct((N, VF), jnp.float32),
            jax.ShapeDtypeStruct((1, 1), jnp.int32),
            jax.ShapeDtypeStruct((1, 1), jnp.int32),
        ],
        scratch_shapes=[
            pltpu.VMEM((1, N), jnp.float32),
            pltpu.VMEM((1, F), jnp.float32),
        ],
    )(x, af_W1, afb1, afW2r, afb2, asW1b, asb1, at1_W, at2_W, at2b,
      v_W, vb, g1, g3)

    a_s = pl.pallas_call(
        _as_body,
        grid=(NT,),
        in_specs=[
            pl.BlockSpec((TILE, F), lambda i: (i, 0)),
            _full((1, F)), _full((1, F)), _full((1, 1)),
            _full((1, N)), _smem11(),
        ],
        out_specs=_smem11(),
        out_shape=jax.ShapeDtypeStruct((1, 1), jnp.int32),
        scratch_shapes=[pltpu.VMEM((1, N), jnp.float32)],
    )(base2, row0, asW2r, asb2, g2, a_f)

    smem = pl.BlockSpec(memory_space=pltpu.SMEM)
    actor = pl.pallas_call(
        _actor_body,
        in_specs=[smem, smem, smem],
        out_shape=jax.ShapeDtypeStruct((1, ACT), jnp.float32),
    )(a_f, a_s, a_t)

    return (actor, critic.reshape(1, N, VF))
